# Initial kernel scaffold; baseline (speedup 1.0000x reference)
import jax
import jax.numpy as jnp
from jax.experimental import pallas as pl


def kernel(gate, codebook, scales, zeros):
    raise NotImplementedError("write your pallas kernel here")


# speedup vs baseline: 7.9906x; 7.9906x over previous
import functools

import jax
import jax.numpy as jnp
from jax.experimental import pallas as pl
from jax.experimental.pallas import tpu as pltpu

_NUM_LUT = 3
_NV = 128
_OUT_F = 2048
_LUT = 16
_VEC = 16
_VPG = 8
_NG = 16
_R = _OUT_F // 8


def _body(gate_ref, cb_ref, sc_ref, zr_ref, out_ref):
    seg = jax.lax.broadcasted_iota(jnp.int32, (_R, 128), 1) % _LUT
    neg = jnp.float32(-3.0e38)

    li = jax.lax.broadcasted_iota(jnp.int32, (128, 128), 0)
    ci = jax.lax.broadcasted_iota(jnp.int32, (128, 128), 1)
    bdmask = (li // _LUT) == (ci // _LUT)

    ai = jax.lax.broadcasted_iota(jnp.int32, (_VPG, 128), 0)
    cj = jax.lax.broadcasted_iota(jnp.int32, (_VPG, 128), 1)
    e8 = jnp.where(cj // _LUT == ai, 1.0, 0.0).astype(jnp.float32)

    s128 = jax.lax.dot(sc_ref[0], e8, precision=jax.lax.Precision.HIGHEST)
    z128 = jax.lax.dot(zr_ref[0], e8, precision=jax.lax.Precision.HIGHEST)

    for vv in range(8):
        ohs = []
        bds = []
        for l in range(_NUM_LUT):
            g = gate_ref[l, vv]
            x = g
            for s in (1, 2, 4, 8):
                yf = pltpu.roll(x, 128 - s, 1)
                x = jnp.maximum(x, jnp.where(seg < _LUT - s, yf, neg))
                yb = pltpu.roll(x, s, 1)
                x = jnp.maximum(x, jnp.where(seg >= s, yb, neg))
            ohs.append((g == x).astype(jnp.float32))
            cb = cb_ref[l, vv]
            bds.append(jnp.where(bdmask, jnp.tile(cb, (8, 8)), 0.0))
        oh = jnp.concatenate(ohs, axis=1)
        bd = jnp.concatenate(bds, axis=0)
        w = jax.lax.dot(oh, bd)
        out_ref[vv] = (w - z128) * s128


@jax.jit
def kernel(gate, codebook, scales, zeros):
    gv = gate.reshape(_NUM_LUT, _NV, _R, 128)
    st = scales.T.reshape(_NG, _R, _VPG)
    zt = zeros.astype(jnp.float32).T.reshape(_NG, _R, _VPG)

    res = pl.pallas_call(
        _body,
        grid=(_NG,),
        in_specs=[
            pl.BlockSpec((_NUM_LUT, _VPG, _R, 128), lambda g: (0, g, 0, 0)),
            pl.BlockSpec((_NUM_LUT, _VPG, _LUT, _VEC), lambda g: (0, g, 0, 0)),
            pl.BlockSpec((1, _R, _VPG), lambda g: (g, 0, 0)),
            pl.BlockSpec((1, _R, _VPG), lambda g: (g, 0, 0)),
        ],
        out_specs=pl.BlockSpec((_VPG, _R, 128), lambda g: (g, 0, 0)),
        out_shape=jax.ShapeDtypeStruct((_NV, _R, 128), jnp.float32),
    )(gv, codebook, st, zt)

    return res.reshape(_NV, _OUT_F, _VEC).transpose(1, 0, 2).reshape(_OUT_F, _NV * _VEC)
